# Initial kernel scaffold; baseline (speedup 1.0000x reference)
import jax
import jax.numpy as jnp
from jax.experimental import pallas as pl


def kernel(x, weight):
    raise NotImplementedError("write your pallas kernel here")


# speedup vs baseline: 725.1770x; 725.1770x over previous
import functools

import jax
import jax.numpy as jnp
from jax import lax
from jax.experimental import pallas as pl
from jax.experimental.pallas import tpu as pltpu
from jax.experimental.pallas import tpu_sc as plsc

VMIN = -1.0
VMAX = 1.0
NUM_WEIGHTS = 63
INV_DPW = (NUM_WEIGHTS - 1) / (VMAX - VMIN)

TOT = 8 * 4096 * 1024
NCORES = 2
NSUB = 16
NWORK = NCORES * NSUB
PER_W = TOT // NWORK
CH = 8192
NCHUNK = PER_W // CH
NVEC = CH // 16
LANES = 16


def _body(x_hbm, w_hbm, out_hbm, lut_v, in_v, out_v,
          in_sem0, in_sem1, out_sem0, out_sem1):
    wid = lax.axis_index("s") * NCORES + lax.axis_index("c")
    base = wid * PER_W

    in_sems = (in_sem0, in_sem1)
    out_sems = (out_sem0, out_sem1)

    pltpu.sync_copy(w_hbm, lut_v)

    def start_in(g, b):
        pltpu.async_copy(x_hbm.at[pl.ds(base + g * CH, CH)], in_v.at[b],
                         in_sems[b])

    def wait_in(g, b):
        pltpu.make_async_copy(x_hbm.at[pl.ds(base + g * CH, CH)],
                              in_v.at[b], in_sems[b]).wait()

    def start_out(g, b):
        pltpu.async_copy(out_v.at[b], out_hbm.at[pl.ds(base + g * CH, CH)],
                         out_sems[b])

    def wait_out(g, b):
        pltpu.make_async_copy(out_v.at[b],
                              out_hbm.at[pl.ds(base + g * CH, CH)],
                              out_sems[b]).wait()

    def compute(b):
        def vec(i, _):
            x = in_v[b, pl.ds(i * LANES, LANES)]
            nd = (x - VMIN) * INV_DPW
            nd = jnp.minimum(jnp.maximum(nd, 0.0), float(NUM_WEIGHTS - 1))
            li = nd.astype(jnp.int32)
            li = jnp.minimum(li, NUM_WEIGHTS - 2)
            f = nd - li.astype(jnp.float32)
            w_lo = plsc.load_gather(lut_v, [li])
            w_hi = plsc.load_gather(lut_v, [li + 1])
            out_v[b, pl.ds(i * LANES, LANES)] = w_lo + f * (w_hi - w_lo)
            return 0
        lax.fori_loop(0, NVEC, vec, 0)

    start_in(0, 0)

    def step(i, _):
        for b in (0, 1):
            g = 2 * i + b

            @pl.when(g + 1 < NCHUNK)
            def _():
                start_in(g + 1, (b + 1) % 2)

            wait_in(g, b)

            @pl.when(g >= 2)
            def _():
                wait_out(g - 2, b)

            compute(b)
            start_out(g, b)
        return 0

    lax.fori_loop(0, NCHUNK // 2, step, 0)

    for b in (0, 1):
        wait_out(NCHUNK - 2 + b, b)


_mesh = plsc.VectorSubcoreMesh(core_axis_name="c", subcore_axis_name="s")

_act = functools.partial(
    pl.kernel,
    out_type=jax.ShapeDtypeStruct((TOT,), jnp.float32),
    mesh=_mesh,
    compiler_params=pltpu.CompilerParams(needs_layout_passes=False),
    scratch_types=[
        pltpu.VMEM((64,), jnp.float32),
        pltpu.VMEM((2, CH), jnp.float32),
        pltpu.VMEM((2, CH), jnp.float32),
        pltpu.SemaphoreType.DMA,
        pltpu.SemaphoreType.DMA,
        pltpu.SemaphoreType.DMA,
        pltpu.SemaphoreType.DMA,
    ],
)(_body)


@jax.jit
def kernel(x, weight):
    w64 = jnp.concatenate([weight, weight[-1:]])
    y = _act(x.reshape(TOT), w64)
    return y.reshape(x.shape)


# speedup vs baseline: 1239.0598x; 1.7086x over previous
import functools

import jax
import jax.numpy as jnp
from jax import lax
from jax.experimental import pallas as pl
from jax.experimental.pallas import tpu as pltpu
from jax.experimental.pallas import tpu_sc as plsc

VMIN = -1.0
VMAX = 1.0
NUM_WEIGHTS = 63
INV_DPW = (NUM_WEIGHTS - 1) / (VMAX - VMIN)

TOT = 8 * 4096 * 1024
NCORES = 2
NSUB = 16
NWORK = NCORES * NSUB
PER_W = TOT // NWORK
CH = 8192
NCHUNK = PER_W // CH
NVEC = CH // 16
LANES = 16


def _body(x_hbm, w_hbm, out_hbm, lut_v, in_v, out_v,
          in_sem0, in_sem1, out_sem0, out_sem1):
    wid = lax.axis_index("s") * NCORES + lax.axis_index("c")
    base = wid * PER_W

    in_sems = (in_sem0, in_sem1)
    out_sems = (out_sem0, out_sem1)

    pltpu.sync_copy(w_hbm, lut_v)

    def start_in(g, b):
        pltpu.async_copy(x_hbm.at[pl.ds(base + g * CH, CH)], in_v.at[b],
                         in_sems[b])

    def wait_in(g, b):
        pltpu.make_async_copy(x_hbm.at[pl.ds(base + g * CH, CH)],
                              in_v.at[b], in_sems[b]).wait()

    def start_out(g, b):
        pltpu.async_copy(out_v.at[b], out_hbm.at[pl.ds(base + g * CH, CH)],
                         out_sems[b])

    def wait_out(g, b):
        pltpu.make_async_copy(out_v.at[b],
                              out_hbm.at[pl.ds(base + g * CH, CH)],
                              out_sems[b]).wait()

    def compute(b):
        @plsc.parallel_loop(0, NVEC, 1, unroll=8)
        def _(i):
            x = in_v[b, pl.ds(i * LANES, LANES)]
            nd = (x - VMIN) * INV_DPW
            nd = jnp.minimum(jnp.maximum(nd, 0.0), float(NUM_WEIGHTS - 1))
            li = nd.astype(jnp.int32)
            li = jnp.minimum(li, NUM_WEIGHTS - 2)
            f = nd - li.astype(jnp.float32)
            w_lo = plsc.load_gather(lut_v, [li])
            w_hi = plsc.load_gather(lut_v, [li + 1])
            out_v[b, pl.ds(i * LANES, LANES)] = w_lo + f * (w_hi - w_lo)

    start_in(0, 0)

    def step(i, _):
        for b in (0, 1):
            g = 2 * i + b

            @pl.when(g + 1 < NCHUNK)
            def _():
                start_in(g + 1, (b + 1) % 2)

            wait_in(g, b)

            @pl.when(g >= 2)
            def _():
                wait_out(g - 2, b)

            compute(b)
            start_out(g, b)
        return 0

    lax.fori_loop(0, NCHUNK // 2, step, 0)

    for b in (0, 1):
        wait_out(NCHUNK - 2 + b, b)


_mesh = plsc.VectorSubcoreMesh(core_axis_name="c", subcore_axis_name="s")

_act = functools.partial(
    pl.kernel,
    out_type=jax.ShapeDtypeStruct((TOT,), jnp.float32),
    mesh=_mesh,
    compiler_params=pltpu.CompilerParams(needs_layout_passes=False),
    scratch_types=[
        pltpu.VMEM((64,), jnp.float32),
        pltpu.VMEM((2, CH), jnp.float32),
        pltpu.VMEM((2, CH), jnp.float32),
        pltpu.SemaphoreType.DMA,
        pltpu.SemaphoreType.DMA,
        pltpu.SemaphoreType.DMA,
        pltpu.SemaphoreType.DMA,
    ],
)(_body)


@jax.jit
def kernel(x, weight):
    w64 = jnp.concatenate([weight, weight[-1:]])
    y = _act(x.reshape(TOT), w64)
    return y.reshape(x.shape)
